# Initial kernel scaffold; baseline (speedup 1.0000x reference)
import jax
import jax.numpy as jnp
from jax.experimental import pallas as pl


def kernel(x):
    raise NotImplementedError("write your pallas kernel here")


# speedup vs baseline: 11.9174x; 11.9174x over previous
import functools

import jax
import jax.numpy as jnp
from jax import lax
from jax.experimental import pallas as pl
from jax.experimental.pallas import tpu as pltpu
from jax.experimental.pallas import tpu_sc as plsc

K_PER_ROW = 64

_NC, _NS, _L = 2, 16, 16
_NW = _NC * _NS

_BINS1 = 1 << 16
_BINS2 = 1 << 15
_W = 16384


def _zero_i32(ref, n):
  def body(i, _):
    ref[pl.ds(i * _L, _L)] = jnp.zeros((_L,), jnp.int32)
    return 0
  lax.fori_loop(0, n // _L, body, 0)


def _keys_from(v):
  bits = lax.bitcast_convert_type(v, jnp.int32)
  return lax.bitwise_and(bits, jnp.int32(0x7FFFFFFF))


def _sc_hist_kernel(n_elems, level2):
  ew = n_elems // _NW
  nwin = ew // _W
  assert ew % _W == 0
  bins = _BINS2 if level2 else _BINS1

  mesh = plsc.VectorSubcoreMesh(
      core_axis_name="c", subcore_axis_name="s",
      num_cores=_NC, num_subcores=_NS)
  scratch = [
      pltpu.VMEM((2, _W), jnp.float32),
      pltpu.VMEM((bins,), jnp.int32),
      pltpu.SemaphoreType.DMA,
      pltpu.SemaphoreType.DMA,
  ]
  if level2:
    scratch.insert(0, pltpu.VMEM((_L,), jnp.int32))

  def body_l1(x_hbm, h_hbm, inbuf, hist, sem0, sem1):
    _run(x_hbm, h_hbm, inbuf, hist, (sem0, sem1), None)

  def body_l2(x_hbm, b_hbm, h_hbm, bbuf, inbuf, hist, sem0, sem1):
    pltpu.sync_copy(b_hbm, bbuf)
    bvec = bbuf[...]
    _run(x_hbm, h_hbm, inbuf, hist, (sem0, sem1), bvec)

  def _run(x_hbm, h_hbm, inbuf, hist, sems, bvec):
    wid = lax.axis_index("s") * _NC + lax.axis_index("c")
    base = wid * ew
    _zero_i32(hist, bins)

    def accum(b):
      def step(j, _):
        v = inbuf[b, pl.ds(j * _L, _L)]
        key = _keys_from(v)
        if bvec is None:
          idx = lax.shift_right_logical(key, 15)
          cnt, last = plsc.scan_count(idx)
        else:
          hi = lax.shift_right_logical(key, 15)
          idx = lax.bitwise_and(key, jnp.int32(0x7FFF))
          cnt, last = plsc.scan_count(idx, mask=hi == bvec)
        plsc.addupdate_scatter(hist, [idx], cnt.astype(jnp.int32), mask=last)
        return 0
      lax.fori_loop(0, _W // _L, step, 0)

    cps = [None, None]
    cps[0] = pltpu.async_copy(x_hbm.at[pl.ds(base, _W)], inbuf.at[0], sems[0])
    for w in range(nwin):
      b = w % 2
      nb = (w + 1) % 2
      if w + 1 < nwin:
        cps[nb] = pltpu.async_copy(
            x_hbm.at[pl.ds(base + (w + 1) * _W, _W)], inbuf.at[nb], sems[nb])
      cps[b].wait()
      accum(b)
    pltpu.sync_copy(hist, h_hbm.at[pl.ds(wid * bins, bins)])

  body = body_l2 if level2 else body_l1
  return pl.kernel(
      body,
      out_type=jax.ShapeDtypeStruct((_NW * bins,), jnp.int32),
      mesh=mesh,
      scratch_types=scratch,
      compiler_params=pltpu.CompilerParams(needs_layout_passes=False),
  )


def _suffix_excl(h):
  rows, cols = h.shape
  jc = lax.broadcasted_iota(jnp.int32, (cols, cols), 0)
  kc = lax.broadcasted_iota(jnp.int32, (cols, cols), 1)
  mcol = (jc > kc).astype(jnp.float32)
  in_row = jnp.dot(h, mcol, preferred_element_type=jnp.float32)
  rowtot = jnp.sum(h, axis=1, keepdims=True)
  jr = lax.broadcasted_iota(jnp.int32, (rows, rows), 0)
  kr = lax.broadcasted_iota(jnp.int32, (rows, rows), 1)
  mrow = (kr > jr).astype(jnp.float32)
  rows_after = jnp.dot(mrow, rowtot, preferred_element_type=jnp.float32)
  return in_row + rows_after


def _select_bin(h, above):
  k = jnp.float32(_K_TOTAL)
  cond = (above < k) & (above + h >= k)
  rows, cols = h.shape
  ri = lax.broadcasted_iota(jnp.int32, (rows, cols), 0)
  ci = lax.broadcasted_iota(jnp.int32, (rows, cols), 1)
  binidx = (ri * cols + ci).astype(jnp.float32)
  b = jnp.sum(jnp.where(cond, binidx, 0.0))
  c = jnp.sum(jnp.where(cond, above, 0.0))
  return b, c


_K_TOTAL = None


def _scan1_body(h_ref, out_ref):
  h = jnp.sum(h_ref[...].astype(jnp.float32), axis=0)
  above = _suffix_excl(h)
  b, c = _select_bin(h, above)
  row = lax.broadcasted_iota(jnp.int32, (8, 128), 0)
  col = lax.broadcasted_iota(jnp.int32, (8, 128), 1)
  o = jnp.where((row == 0) & (col == 0), b.astype(jnp.int32), 0)
  o = o + jnp.where((row == 0) & (col == 1), c.astype(jnp.int32), 0)
  out_ref[...] = o


def _scan2_body(h_ref, prior_ref, out_ref):
  h = jnp.sum(h_ref[...].astype(jnp.float32), axis=0)
  c_base = prior_ref[0, 1].astype(jnp.float32)
  b_hi = prior_ref[0, 0]
  above = _suffix_excl(h) + c_base
  t_lo, c_sel = _select_bin(h, above)
  t = lax.shift_left(b_hi, 15) | t_lo.astype(jnp.int32)
  r = jnp.float32(_K_TOTAL) - c_sel
  row = lax.broadcasted_iota(jnp.int32, (8, 128), 0)
  col = lax.broadcasted_iota(jnp.int32, (8, 128), 1)
  o = jnp.where((row == 0) & (col == 0), t, 0)
  o = o + jnp.where((row == 0) & (col == 1), r.astype(jnp.int32), 0)
  out_ref[...] = o


def _mask_body(x_ref, td_ref, out_ref, consumed):
  g = pl.program_id(0)

  @pl.when(g == 0)
  def _():
    consumed[0] = 0

  xb = x_ref[...]
  key = _keys_from(xb)
  t = td_ref[0, 0]
  r = td_ref[0, 1]
  gt = key > t
  eq = key == t
  c = jnp.sum(eq.astype(jnp.int32))
  rem = r - consumed[0]
  take_none = (c == 0) | (rem <= 0)
  take_all = jnp.logical_not(take_none) & (c <= rem)
  take_some = jnp.logical_not(take_none) & (c > rem)

  @pl.when(take_none)
  def _():
    out_ref[...] = jnp.where(gt, xb, 0.0)

  @pl.when(take_all)
  def _():
    out_ref[...] = jnp.where(gt | eq, xb, 0.0)

  @pl.when(take_some)
  def _():
    rows, cols = eq.shape
    e = eq.astype(jnp.float32)
    jc = lax.broadcasted_iota(jnp.int32, (cols, cols), 0)
    kc = lax.broadcasted_iota(jnp.int32, (cols, cols), 1)
    incl = (jc <= kc).astype(jnp.float32)
    pr_row = jnp.dot(e, incl, preferred_element_type=jnp.float32)
    rowtot = jnp.sum(e, axis=1, keepdims=True)
    jr = lax.broadcasted_iota(jnp.int32, (rows, rows), 0)
    kr = lax.broadcasted_iota(jnp.int32, (rows, rows), 1)
    strict = (kr < jr).astype(jnp.float32)
    rows_before = jnp.dot(strict, rowtot, preferred_element_type=jnp.float32)
    prefix = pr_row + rows_before
    keep = eq & (prefix <= rem.astype(jnp.float32))
    out_ref[...] = jnp.where(gt | keep, xb, 0.0)

  consumed[0] = consumed[0] + c


def kernel(x):
  global _K_TOTAL
  batch, feat = x.shape
  n = batch * feat
  _K_TOTAL = K_PER_ROW * batch

  x_flat = x.reshape(-1)

  hist1 = _sc_hist_kernel(n, level2=False)(x_flat)
  h1 = hist1.reshape(_NW, _BINS1 // 128, 128)

  bd = pl.pallas_call(
      _scan1_body,
      out_shape=jax.ShapeDtypeStruct((8, 128), jnp.int32),
  )(h1)

  b_vec = jnp.broadcast_to(bd[0, 0], (_L,)).astype(jnp.int32)

  hist2 = _sc_hist_kernel(n, level2=True)(x_flat, b_vec)
  h2 = hist2.reshape(_NW, _BINS2 // 128, 128)

  td = pl.pallas_call(
      _scan2_body,
      in_specs=[
          pl.BlockSpec(memory_space=pltpu.VMEM),
          pl.BlockSpec(memory_space=pltpu.SMEM),
      ],
      out_shape=jax.ShapeDtypeStruct((8, 128), jnp.int32),
  )(h2, bd[:1, :2])

  rows = n // 1024
  blk = 512
  x2d = x_flat.reshape(rows, 1024)
  out = pl.pallas_call(
      _mask_body,
      grid=(rows // blk,),
      in_specs=[
          pl.BlockSpec((blk, 1024), lambda g: (g, 0)),
          pl.BlockSpec(memory_space=pltpu.SMEM),
      ],
      out_specs=pl.BlockSpec((blk, 1024), lambda g: (g, 0)),
      out_shape=jax.ShapeDtypeStruct((rows, 1024), jnp.float32),
      scratch_shapes=[pltpu.SMEM((1,), jnp.int32)],
      compiler_params=pltpu.CompilerParams(
          dimension_semantics=("arbitrary",)),
  )(x2d, td[:1, :2])

  return out.reshape(batch, feat)


# speedup vs baseline: 36.5989x; 3.0711x over previous
import functools

import jax
import jax.numpy as jnp
from jax import lax
from jax.experimental import pallas as pl
from jax.experimental.pallas import tpu as pltpu
from jax.experimental.pallas import tpu_sc as plsc

K_PER_ROW = 64

_NC, _NS, _L = 2, 16, 16
_NW = _NC * _NS

_BINS1 = 1 << 16
_BINS2 = 1 << 15
_W = 16384


def _zero_i32(ref, n):
  zeros = jnp.zeros((_L,), jnp.int32)

  @plsc.parallel_loop(0, n // _L, unroll=8)
  def _(i):
    ref[pl.ds(i * _L, _L)] = zeros


def _keys_from(v):
  bits = lax.bitcast_convert_type(v, jnp.int32)
  return lax.bitwise_and(bits, jnp.int32(0x7FFFFFFF))


def _sc_hist_kernel(n_elems, level2):
  ew = n_elems // _NW
  nwin = ew // _W
  assert ew % _W == 0
  bins = _BINS2 if level2 else _BINS1

  mesh = plsc.VectorSubcoreMesh(
      core_axis_name="c", subcore_axis_name="s",
      num_cores=_NC, num_subcores=_NS)
  scratch = [
      pltpu.VMEM((2, _W), jnp.float32),
      pltpu.VMEM((bins,), jnp.int32),
      pltpu.SemaphoreType.DMA,
      pltpu.SemaphoreType.DMA,
  ]
  if level2:
    scratch.insert(0, pltpu.VMEM((_L,), jnp.int32))

  def body_l1(x_hbm, h_hbm, inbuf, hist, sem0, sem1):
    _run(x_hbm, h_hbm, inbuf, hist, (sem0, sem1), None)

  def body_l2(x_hbm, b_hbm, h_hbm, bbuf, inbuf, hist, sem0, sem1):
    pltpu.sync_copy(b_hbm, bbuf)
    bvec = bbuf[...]
    _run(x_hbm, h_hbm, inbuf, hist, (sem0, sem1), bvec)

  def _run(x_hbm, h_hbm, inbuf, hist, sems, bvec):
    wid = lax.axis_index("s") * _NC + lax.axis_index("c")
    base = wid * ew
    _zero_i32(hist, bins)

    ones = jnp.ones((_L,), jnp.int32)

    def accum(b):
      @plsc.parallel_loop(0, _W // _L, unroll=8)
      def _(j):
        v = inbuf[b, pl.ds(j * _L, _L)]
        key = _keys_from(v)
        if bvec is None:
          idx = lax.shift_right_logical(key, 15)
          plsc.addupdate_scatter(hist, [idx], ones)
        else:
          hi = lax.shift_right_logical(key, 15)
          idx = lax.bitwise_and(key, jnp.int32(0x7FFF))
          plsc.addupdate_scatter(hist, [idx], ones, mask=hi == bvec)

    cps = [None, None]
    cps[0] = pltpu.async_copy(x_hbm.at[pl.ds(base, _W)], inbuf.at[0], sems[0])
    for w in range(nwin):
      b = w % 2
      nb = (w + 1) % 2
      if w + 1 < nwin:
        cps[nb] = pltpu.async_copy(
            x_hbm.at[pl.ds(base + (w + 1) * _W, _W)], inbuf.at[nb], sems[nb])
      cps[b].wait()
      accum(b)
    pltpu.sync_copy(hist, h_hbm.at[pl.ds(wid * bins, bins)])

  body = body_l2 if level2 else body_l1
  return pl.kernel(
      body,
      out_type=jax.ShapeDtypeStruct((_NW * bins,), jnp.int32),
      mesh=mesh,
      scratch_types=scratch,
      compiler_params=pltpu.CompilerParams(needs_layout_passes=False),
  )


def _suffix_excl(h):
  rows, cols = h.shape
  jc = lax.broadcasted_iota(jnp.int32, (cols, cols), 0)
  kc = lax.broadcasted_iota(jnp.int32, (cols, cols), 1)
  mcol = (jc > kc).astype(jnp.float32)
  in_row = jnp.dot(h, mcol, preferred_element_type=jnp.float32,
                   precision=lax.Precision.HIGHEST)
  rowtot = jnp.sum(h, axis=1, keepdims=True)
  jr = lax.broadcasted_iota(jnp.int32, (rows, rows), 0)
  kr = lax.broadcasted_iota(jnp.int32, (rows, rows), 1)
  mrow = (kr > jr).astype(jnp.float32)
  rows_after = jnp.dot(mrow, rowtot, preferred_element_type=jnp.float32,
                   precision=lax.Precision.HIGHEST)
  return in_row + rows_after


def _select_bin(h, above):
  k = jnp.float32(_K_TOTAL)
  cond = (above < k) & (above + h >= k)
  rows, cols = h.shape
  ri = lax.broadcasted_iota(jnp.int32, (rows, cols), 0)
  ci = lax.broadcasted_iota(jnp.int32, (rows, cols), 1)
  binidx = (ri * cols + ci).astype(jnp.float32)
  b = jnp.sum(jnp.where(cond, binidx, 0.0))
  c = jnp.sum(jnp.where(cond, above, 0.0))
  return b, c


_K_TOTAL = None


def _scan1_body(h_ref, out_ref):
  h = jnp.sum(h_ref[...].astype(jnp.float32), axis=0)
  above = _suffix_excl(h)
  b, c = _select_bin(h, above)
  row = lax.broadcasted_iota(jnp.int32, (8, 128), 0)
  col = lax.broadcasted_iota(jnp.int32, (8, 128), 1)
  o = jnp.where((row == 0) & (col == 0), b.astype(jnp.int32), 0)
  o = o + jnp.where((row == 0) & (col == 1), c.astype(jnp.int32), 0)
  out_ref[...] = o


def _scan2_body(h_ref, prior_ref, out_ref):
  h = jnp.sum(h_ref[...].astype(jnp.float32), axis=0)
  c_base = prior_ref[0, 1].astype(jnp.float32)
  b_hi = prior_ref[0, 0]
  above = _suffix_excl(h) + c_base
  t_lo, c_sel = _select_bin(h, above)
  t = lax.shift_left(b_hi, 15) | t_lo.astype(jnp.int32)
  r = jnp.float32(_K_TOTAL) - c_sel
  row = lax.broadcasted_iota(jnp.int32, (8, 128), 0)
  col = lax.broadcasted_iota(jnp.int32, (8, 128), 1)
  o = jnp.where((row == 0) & (col == 0), t, 0)
  o = o + jnp.where((row == 0) & (col == 1), r.astype(jnp.int32), 0)
  out_ref[...] = o


def _mask_body(x_ref, td_ref, out_ref, consumed):
  g = pl.program_id(0)

  @pl.when(g == 0)
  def _():
    consumed[0] = 0

  xb = x_ref[...]
  key = _keys_from(xb)
  t = td_ref[0, 0]
  r = td_ref[0, 1]
  gt = key > t
  eq = key == t
  c = jnp.sum(eq.astype(jnp.int32))
  rem = r - consumed[0]
  take_none = (c == 0) | (rem <= 0)
  take_all = jnp.logical_not(take_none) & (c <= rem)
  take_some = jnp.logical_not(take_none) & (c > rem)

  @pl.when(take_none)
  def _():
    out_ref[...] = jnp.where(gt, xb, 0.0)

  @pl.when(take_all)
  def _():
    out_ref[...] = jnp.where(gt | eq, xb, 0.0)

  @pl.when(take_some)
  def _():
    rows, cols = eq.shape
    e = eq.astype(jnp.float32)
    jc = lax.broadcasted_iota(jnp.int32, (cols, cols), 0)
    kc = lax.broadcasted_iota(jnp.int32, (cols, cols), 1)
    incl = (jc <= kc).astype(jnp.float32)
    pr_row = jnp.dot(e, incl, preferred_element_type=jnp.float32,
                   precision=lax.Precision.HIGHEST)
    rowtot = jnp.sum(e, axis=1, keepdims=True)
    jr = lax.broadcasted_iota(jnp.int32, (rows, rows), 0)
    kr = lax.broadcasted_iota(jnp.int32, (rows, rows), 1)
    strict = (kr < jr).astype(jnp.float32)
    rows_before = jnp.dot(strict, rowtot, preferred_element_type=jnp.float32,
                   precision=lax.Precision.HIGHEST)
    prefix = pr_row + rows_before
    keep = eq & (prefix <= rem.astype(jnp.float32))
    out_ref[...] = jnp.where(gt | keep, xb, 0.0)

  consumed[0] = consumed[0] + c


def kernel(x):
  global _K_TOTAL
  batch, feat = x.shape
  n = batch * feat
  _K_TOTAL = K_PER_ROW * batch

  x_flat = x.reshape(-1)

  hist1 = _sc_hist_kernel(n, level2=False)(x_flat)
  h1 = hist1.reshape(_NW, _BINS1 // 128, 128)

  bd = pl.pallas_call(
      _scan1_body,
      out_shape=jax.ShapeDtypeStruct((8, 128), jnp.int32),
  )(h1)

  b_vec = jnp.broadcast_to(bd[0, 0], (_L,)).astype(jnp.int32)

  hist2 = _sc_hist_kernel(n, level2=True)(x_flat, b_vec)
  h2 = hist2.reshape(_NW, _BINS2 // 128, 128)

  td = pl.pallas_call(
      _scan2_body,
      in_specs=[
          pl.BlockSpec(memory_space=pltpu.VMEM),
          pl.BlockSpec(memory_space=pltpu.SMEM),
      ],
      out_shape=jax.ShapeDtypeStruct((8, 128), jnp.int32),
  )(h2, bd[:1, :2])

  rows = n // 1024
  blk = 512
  x2d = x_flat.reshape(rows, 1024)
  out = pl.pallas_call(
      _mask_body,
      grid=(rows // blk,),
      in_specs=[
          pl.BlockSpec((blk, 1024), lambda g: (g, 0)),
          pl.BlockSpec(memory_space=pltpu.SMEM),
      ],
      out_specs=pl.BlockSpec((blk, 1024), lambda g: (g, 0)),
      out_shape=jax.ShapeDtypeStruct((rows, 1024), jnp.float32),
      scratch_shapes=[pltpu.SMEM((1,), jnp.int32)],
      compiler_params=pltpu.CompilerParams(
          dimension_semantics=("arbitrary",)),
  )(x2d, td[:1, :2])

  return out.reshape(batch, feat)
